# baseline (speedup 1.0000x reference)
import jax
import jax.numpy as jnp
from jax.experimental import pallas as pl
from jax.experimental.pallas import tpu as pltpu

N, E = 10000, 2000
IN, HID, OUT = 128, 128, 64
NEG = 0.2
RB = 200
NRB = N // RB
BF = jnp.bfloat16
F32 = jnp.float32


def _lrelu(x):
    return jnp.where(x > 0, x, NEG * x)


def _dotT(a, b):
    return jax.lax.dot_general(a, b, (((0,), (0,)), ((), ())),
                               preferred_element_type=F32)


def _dot(a, b):
    return jnp.dot(a, b, preferred_element_type=F32)


def _edge_accum(k, x, Hb, W_ref, W2_ref, b_ref, ctx_ref, a_ref, a2_ref, do,
                Pn_ref, Pd_ref, m_ref, mslot):
    @pl.when(k == 0)
    def _():
        Pn_ref[...] = jnp.zeros_like(Pn_ref)
        Pd_ref[...] = jnp.zeros_like(Pd_ref)
        m_ref[0, mslot] = -1e30

    xt = _dot(x, W_ref[...]) + b_ref[...]
    sctx = _dot(ctx_ref[...], a_ref[0:do, :])
    v1 = _dot(W2_ref[...], a_ref[do:2 * do, :])
    v2 = _dot(W2_ref[...], a2_ref[0:do, :])
    s1 = _lrelu(sctx + _dot(x, v1))

    m_old = m_ref[0, mslot]
    bmax = jnp.max(s1)
    m_new = jnp.maximum(m_old, bmax)
    w = jnp.exp(s1 - m_new)
    Dn = _dotT(Hb, (xt * w).astype(BF))
    Dd = _dotT(Hb, jnp.broadcast_to(w, (w.shape[0], 8)).astype(BF))

    @pl.when(bmax > m_old)
    def _():
        alpha = jnp.exp(m_old - m_new)
        Pn_ref[...] = alpha * Pn_ref[...] + Dn
        Pd_ref[...] = alpha * Pd_ref[...] + Dd
        m_ref[0, mslot] = bmax

    @pl.when(bmax <= m_old)
    def _():
        Pn_ref[...] += Dn
        Pd_ref[...] += Dd
    return v2


def _node_attn(k, Hb, s2, Pn_ref, Pd_ref, W3_ref, a2_ref, do,
               ebx_ref, c1_ref, c2_ref, m_ref, mslot):
    @pl.when(k == 0)
    def _():
        edge = Pn_ref[...] / Pd_ref[:, 0:1]
        ebx_ref[...] = jnp.concatenate(
            [edge.astype(BF), jnp.ones((E, 8), BF)], axis=1)
        w3a = _dot(W3_ref[...], a2_ref[do:2 * do, :])
        s3 = jax.lax.dot_general(
            w3a, edge, (((0,), (1,)), ((), ())),
            preferred_element_type=F32)
        m3 = jnp.max(s3)
        t = s3 - m3
        c1_ref[...] = jnp.exp(t).astype(BF)
        c2_ref[...] = jnp.exp(NEG * t).astype(BF)
        m_ref[0, mslot] = m3

    q = jnp.exp((1.0 - NEG) * (s2 + m_ref[0, mslot])).astype(BF)
    B = Hb * jnp.maximum(q * c1_ref[...], c2_ref[...])
    nd = _dot(B, ebx_ref[...])
    return nd[:, :do] / nd[:, do:do + 1]


def _node_post(node, x, res_ref, g_ref, be_ref, al_ref):
    y = jnp.where(node > 0, node, jnp.exp(jnp.minimum(node, 0.0)) - 1.0)
    mu = jnp.mean(y, axis=1, keepdims=True)
    c = y - mu
    v = jnp.mean(c * c, axis=1, keepdims=True)
    xn = c * jax.lax.rsqrt(v + 1e-5) * g_ref[...] + be_ref[...]
    al = al_ref[0, 0]
    return al * xn + (1.0 - al) * _dot(x, res_ref[...])


def _hgat_kernel(x_ref, H_ref,
                 W0_ref, W20_ref, W30_ref, b0_ref, a0_ref, a20_ref,
                 ctx0_ref, res0_ref, g0_ref, be0_ref, al0_ref,
                 W1_ref, W21_ref, W31_ref, b1_ref, a1_ref, a21_ref,
                 ctx1_ref, res1_ref, g1_ref, be1_ref, al1_ref,
                 cW1_ref, cb1_ref, cW2_ref, cb2_ref,
                 out_ref,
                 Hb_ref, x1_ref, Pn0_ref, Pd0_ref, Pn1_ref, Pd1_ref,
                 ebx0_ref, ebx1_ref, c10_ref, c20_ref, c11_ref, c21_ref,
                 m_ref):
    p = pl.program_id(0)
    k = pl.program_id(1)
    rows = pl.ds(k * RB, RB)

    @pl.when(p == 0)
    def _():
        Hb = H_ref[...].astype(BF)
        Hb_ref[rows, :] = Hb
        _edge_accum(k, x_ref[...], Hb, W0_ref, W20_ref, b0_ref, ctx0_ref,
                    a0_ref, a20_ref, HID, Pn0_ref, Pd0_ref, m_ref, 0)

    @pl.when(p == 1)
    def _():
        Hb = Hb_ref[rows, :]
        x = x_ref[...]
        v2 = _dot(W20_ref[...], a20_ref[0:HID, :])
        s2 = _dot(x, v2)
        node = _node_attn(k, Hb, s2, Pn0_ref, Pd0_ref, W30_ref, a20_ref,
                          HID, ebx0_ref, c10_ref, c20_ref, m_ref, 2)
        x1 = _node_post(node, x, res0_ref, g0_ref, be0_ref, al0_ref)
        x1_ref[rows, :] = x1
        _edge_accum(k, x1, Hb, W1_ref, W21_ref, b1_ref, ctx1_ref,
                    a1_ref, a21_ref, OUT, Pn1_ref, Pd1_ref, m_ref, 1)

    @pl.when(p == 2)
    def _():
        Hb = Hb_ref[rows, :]
        x1 = x1_ref[rows, :]
        v2 = _dot(W21_ref[...], a21_ref[0:OUT, :])
        s2 = _dot(x1, v2)
        node = _node_attn(k, Hb, s2, Pn1_ref, Pd1_ref, W31_ref, a21_ref,
                          OUT, ebx1_ref, c11_ref, c21_ref, m_ref, 3)
        x2 = _node_post(node, x1, res1_ref, g1_ref, be1_ref, al1_ref)
        h = jnp.maximum(_dot(x1, cW1_ref[0:HID, :])
                        + _dot(x2, cW1_ref[HID:HID + OUT, :])
                        + cb1_ref[...], 0.0)
        out_ref[...] = _dot(h, cW2_ref[...]) + cb2_ref[...]


def _full(shape):
    nd = len(shape)
    return pl.BlockSpec(shape, lambda p, k: (0,) * nd)


def kernel(X, H, W0, W2_0, W3_0, b0, a0, a2_0, ctx0, res0, g0, be0, al0,
           W1, W2_1, W3_1, b1, a1, a2_1, ctx1, res1, g1, be1, al1,
           cW1, cb1, cW2, cb2):
    JK = HID + OUT
    out = pl.pallas_call(
        _hgat_kernel,
        grid=(3, NRB),
        in_specs=[
            pl.BlockSpec((RB, IN), lambda p, k: (jnp.where(p <= 1, k, 0), 0)),
            pl.BlockSpec((RB, E), lambda p, k: (jnp.where(p == 0, k, 0), 0)),
            _full((IN, HID)), _full((IN, HID)), _full((HID, HID)),
            _full((1, HID)), _full((2 * HID, 1)), _full((2 * HID, 1)),
            _full((1, HID)), _full((IN, HID)), _full((1, HID)),
            _full((1, HID)), _full((1, 1)),
            _full((HID, OUT)), _full((HID, OUT)), _full((OUT, OUT)),
            _full((1, OUT)), _full((2 * OUT, 1)), _full((2 * OUT, 1)),
            _full((1, OUT)), _full((HID, OUT)), _full((1, OUT)),
            _full((1, OUT)), _full((1, 1)),
            _full((JK, HID)), _full((1, HID)), _full((HID, OUT)),
            _full((1, OUT)),
        ],
        out_specs=pl.BlockSpec((RB, OUT),
                               lambda p, k: (jnp.where(p == 2, k, 0), 0)),
        out_shape=jax.ShapeDtypeStruct((N, OUT), F32),
        scratch_shapes=[
            pltpu.VMEM((N, E), BF),
            pltpu.VMEM((N, HID), F32),
            pltpu.VMEM((E, HID), F32),
            pltpu.VMEM((E, 8), F32),
            pltpu.VMEM((E, OUT), F32),
            pltpu.VMEM((E, 8), F32),
            pltpu.VMEM((E, HID + 8), BF),
            pltpu.VMEM((E, OUT + 8), BF),
            pltpu.VMEM((1, E), BF),
            pltpu.VMEM((1, E), BF),
            pltpu.VMEM((1, E), BF),
            pltpu.VMEM((1, E), BF),
            pltpu.SMEM((1, 8), F32),
        ],
    )(X, H, W0, W2_0, W3_0, b0.reshape(1, HID), a0, a2_0,
      ctx0.reshape(1, HID), res0, g0.reshape(1, HID), be0.reshape(1, HID),
      al0.reshape(1, 1),
      W1, W2_1, W3_1, b1.reshape(1, OUT), a1, a2_1,
      ctx1.reshape(1, OUT), res1, g1.reshape(1, OUT), be1.reshape(1, OUT),
      al1.reshape(1, 1),
      cW1, cb1.reshape(1, HID), cW2, cb2.reshape(1, OUT))
    return out


# speedup vs baseline: 1.3561x; 1.3561x over previous
import jax
import jax.numpy as jnp
from jax.experimental import pallas as pl
from jax.experimental.pallas import tpu as pltpu

N, E = 10000, 2000
IN, HID, OUT = 128, 128, 64
NEG = 0.2
RB = 400
NRB = N // RB
BF = jnp.bfloat16
F32 = jnp.float32


def _lrelu(x):
    return jnp.where(x > 0, x, NEG * x)


def _dotT(a, b):
    return jax.lax.dot_general(a, b, (((0,), (0,)), ((), ())),
                               preferred_element_type=F32)


def _dot(a, b):
    return jnp.dot(a, b, preferred_element_type=F32)


def _rowdot(a, b):
    return jax.lax.dot_general(a, b, (((1,), (1,)), ((), ())),
                               preferred_element_type=F32)


def _edge_accum(k, x, Hb, W_ref, W2_ref, b_ref, ctx_ref, a_ref, a2_ref, do,
                Pn_ref, Pd_ref, m_ref, mslot):
    @pl.when(k == 0)
    def _():
        Pn_ref[...] = jnp.zeros_like(Pn_ref)
        Pd_ref[...] = jnp.zeros_like(Pd_ref)
        m_ref[0, mslot] = -1e30

    xt = _dot(x, W_ref[...]) + b_ref[...]
    sctx = _rowdot(ctx_ref[...], a_ref[:, 0:do])
    v1 = _rowdot(W2_ref[...], a_ref[:, do:2 * do])
    s1 = _lrelu(sctx + _dot(x, v1))

    m_old = m_ref[0, mslot]
    bmax = jnp.max(s1)
    m_new = jnp.maximum(m_old, bmax)
    w = jnp.exp(s1 - m_new)
    Dn = _dotT(Hb, (xt * w).astype(BF))
    Dd = _dotT(Hb, jnp.broadcast_to(w, (w.shape[0], 8)).astype(BF))

    @pl.when(bmax > m_old)
    def _():
        alpha = jnp.exp(m_old - m_new)
        Pn_ref[...] = alpha * Pn_ref[...] + Dn
        Pd_ref[...] = alpha * Pd_ref[...] + Dd
        m_ref[0, mslot] = bmax

    @pl.when(bmax <= m_old)
    def _():
        Pn_ref[...] += Dn
        Pd_ref[...] += Dd


def _node_attn(k, Hb, s2, Pn_ref, Pd_ref, W3_ref, a2_ref, do,
               ebx_ref, c1_ref, c2_ref, m_ref, mslot):
    @pl.when(k == 0)
    def _():
        edge = Pn_ref[...] / Pd_ref[:, 0:1]
        ebx_ref[...] = jnp.concatenate(
            [edge.astype(BF), jnp.ones((E, 8), BF)], axis=1)
        w3a = _rowdot(W3_ref[...], a2_ref[:, do:2 * do])
        s3 = jax.lax.dot_general(
            w3a, edge, (((0,), (1,)), ((), ())),
            preferred_element_type=F32)
        m3 = jnp.max(s3)
        t = s3 - m3
        c1_ref[...] = jnp.exp(t).astype(BF)
        c2_ref[...] = jnp.exp(NEG * t).astype(BF)
        m_ref[0, mslot] = m3

    q = jnp.exp((1.0 - NEG) * (s2 + m_ref[0, mslot])).astype(BF)
    B = Hb * jnp.maximum(q * c1_ref[...], c2_ref[...])
    nd = _dot(B, ebx_ref[...])
    return nd[:, :do] / nd[:, do:do + 1]


def _node_post(node, x, res_ref, g_ref, be_ref, al_ref):
    y = jnp.where(node > 0, node, jnp.exp(jnp.minimum(node, 0.0)) - 1.0)
    mu = jnp.mean(y, axis=1, keepdims=True)
    c = y - mu
    v = jnp.mean(c * c, axis=1, keepdims=True)
    xn = c * jax.lax.rsqrt(v + 1e-5) * g_ref[...] + be_ref[...]
    al = al_ref[0, 0]
    return al * xn + (1.0 - al) * _dot(x, res_ref[...])


def _hgat_kernel(x_ref, H_ref,
                 W0_ref, W20_ref, W30_ref, b0_ref, a0_ref, a20_ref,
                 ctx0_ref, res0_ref, g0_ref, be0_ref, al0_ref,
                 W1_ref, W21_ref, W31_ref, b1_ref, a1_ref, a21_ref,
                 ctx1_ref, res1_ref, g1_ref, be1_ref, al1_ref,
                 cW1_ref, cb1_ref, cW2_ref, cb2_ref,
                 out_ref,
                 Hb_ref, x1_ref, Pn0_ref, Pn1_ref, Pd_ref,
                 ebx0_ref, ebx1_ref, c1_ref, c2_ref,
                 m_ref):
    p = pl.program_id(0)
    k = pl.program_id(1)
    rows = pl.ds(k * RB, RB)

    @pl.when(p == 0)
    def _():
        Hb = H_ref[...].astype(BF)
        Hb_ref[rows, :] = Hb
        _edge_accum(k, x_ref[...], Hb, W0_ref, W20_ref, b0_ref, ctx0_ref,
                    a0_ref, a20_ref, HID, Pn0_ref, Pd_ref, m_ref, 0)

    @pl.when(p == 1)
    def _():
        Hb = Hb_ref[rows, :]
        x = x_ref[...]
        v2 = _rowdot(W20_ref[...], a20_ref[:, 0:HID])
        s2 = _dot(x, v2)
        node = _node_attn(k, Hb, s2, Pn0_ref, Pd_ref, W30_ref, a20_ref,
                          HID, ebx0_ref, c1_ref, c2_ref, m_ref, 2)
        x1 = _node_post(node, x, res0_ref, g0_ref, be0_ref, al0_ref)
        x1_ref[rows, :] = x1.astype(BF)
        _edge_accum(k, x1, Hb, W1_ref, W21_ref, b1_ref, ctx1_ref,
                    a1_ref, a21_ref, OUT, Pn1_ref, Pd_ref, m_ref, 1)

    @pl.when(p == 2)
    def _():
        Hb = Hb_ref[rows, :]
        x1 = x1_ref[rows, :].astype(F32)
        v2 = _rowdot(W21_ref[...], a21_ref[:, 0:OUT])
        s2 = _dot(x1, v2)
        node = _node_attn(k, Hb, s2, Pn1_ref, Pd_ref, W31_ref, a21_ref,
                          OUT, ebx1_ref, c1_ref, c2_ref, m_ref, 3)
        x2 = _node_post(node, x1, res1_ref, g1_ref, be1_ref, al1_ref)
        h = jnp.maximum(_dot(x1, cW1_ref[0:HID, :])
                        + _dot(x2, cW1_ref[HID:HID + OUT, :])
                        + cb1_ref[...], 0.0)
        out_ref[...] = _dot(h, cW2_ref[...]) + cb2_ref[...]


def _full(shape):
    nd = len(shape)
    return pl.BlockSpec(shape, lambda p, k: (0,) * nd)


def kernel(X, H, W0, W2_0, W3_0, b0, a0, a2_0, ctx0, res0, g0, be0, al0,
           W1, W2_1, W3_1, b1, a1, a2_1, ctx1, res1, g1, be1, al1,
           cW1, cb1, cW2, cb2):
    JK = HID + OUT
    out = pl.pallas_call(
        _hgat_kernel,
        grid=(3, NRB),
        in_specs=[
            pl.BlockSpec((RB, IN), lambda p, k: (jnp.where(p <= 1, k, 0), 0)),
            pl.BlockSpec((RB, E), lambda p, k: (jnp.where(p == 0, k, 0), 0)),
            _full((IN, HID)), _full((IN, HID)), _full((HID, HID)),
            _full((1, HID)), _full((1, 2 * HID)), _full((1, 2 * HID)),
            _full((1, HID)), _full((IN, HID)), _full((1, HID)),
            _full((1, HID)), _full((1, 1)),
            _full((HID, OUT)), _full((HID, OUT)), _full((OUT, OUT)),
            _full((1, OUT)), _full((1, 2 * OUT)), _full((1, 2 * OUT)),
            _full((1, OUT)), _full((HID, OUT)), _full((1, OUT)),
            _full((1, OUT)), _full((1, 1)),
            _full((JK, HID)), _full((1, HID)), _full((HID, OUT)),
            _full((1, OUT)),
        ],
        out_specs=pl.BlockSpec((RB, OUT),
                               lambda p, k: (jnp.where(p == 2, k, 0), 0)),
        out_shape=jax.ShapeDtypeStruct((N, OUT), F32),
        scratch_shapes=[
            pltpu.VMEM((N, E), BF),
            pltpu.VMEM((N, HID), BF),
            pltpu.VMEM((E, HID), F32),
            pltpu.VMEM((E, OUT), F32),
            pltpu.VMEM((E, 8), F32),
            pltpu.VMEM((E, HID + 8), BF),
            pltpu.VMEM((E, OUT + 8), BF),
            pltpu.VMEM((1, E), BF),
            pltpu.VMEM((1, E), BF),
            pltpu.SMEM((1, 8), F32),
        ],
    )(X, H, W0, W2_0, W3_0, b0.reshape(1, HID), a0.reshape(1, 2 * HID),
      a2_0.reshape(1, 2 * HID),
      ctx0.reshape(1, HID), res0, g0.reshape(1, HID), be0.reshape(1, HID),
      al0.reshape(1, 1),
      W1, W2_1, W3_1, b1.reshape(1, OUT), a1.reshape(1, 2 * OUT),
      a2_1.reshape(1, 2 * OUT),
      ctx1.reshape(1, OUT), res1, g1.reshape(1, OUT), be1.reshape(1, OUT),
      al1.reshape(1, 1),
      cW1, cb1.reshape(1, HID), cW2, cb2.reshape(1, OUT))
    return out


# speedup vs baseline: 1.6754x; 1.2355x over previous
import jax
import jax.numpy as jnp
from jax.experimental import pallas as pl
from jax.experimental.pallas import tpu as pltpu

N, E = 10000, 2000
IN, HID, OUT = 128, 128, 64
NEG = 0.2
RB = 1000
NRB = N // RB
BF = jnp.bfloat16
F32 = jnp.float32


def _lrelu(x):
    return jnp.where(x > 0, x, NEG * x)


def _dotT(a, b):
    return jax.lax.dot_general(a, b, (((0,), (0,)), ((), ())),
                               preferred_element_type=F32)


def _dot(a, b):
    return jnp.dot(a, b, preferred_element_type=F32)


def _rowdot(a, b):
    return jax.lax.dot_general(a, b, (((1,), (1,)), ((), ())),
                               preferred_element_type=F32)


def _edge_accum(k, x, Hb, W_ref, W2_ref, b_ref, ctx_ref, a_ref, a2_ref, do,
                Pn_ref, m_ref, mslot):
    @pl.when(k == 0)
    def _():
        Pn_ref[...] = jnp.zeros_like(Pn_ref)
        m_ref[0, mslot] = -1e30

    xt = _dot(x, W_ref[...]) + b_ref[...]
    sctx = _rowdot(ctx_ref[...], a_ref[:, 0:do])
    v1 = _rowdot(W2_ref[...], a_ref[:, do:2 * do])
    s1 = _lrelu(sctx + _dot(x, v1))

    m_old = m_ref[0, mslot]
    bmax = jnp.max(s1)
    m_new = jnp.maximum(m_old, bmax)
    w = jnp.exp(s1 - m_new)
    V = jnp.concatenate(
        [(xt * w).astype(BF),
         jnp.broadcast_to(w, (w.shape[0], 8)).astype(BF)], axis=1)
    D = _dotT(Hb, V)

    @pl.when(bmax > m_old)
    def _():
        alpha = jnp.exp(m_old - m_new)
        Pn_ref[...] = alpha * Pn_ref[...] + D
        m_ref[0, mslot] = bmax

    @pl.when(bmax <= m_old)
    def _():
        Pn_ref[...] += D


def _node_attn(k, Hb, s2, Pn_ref, W3_ref, a2_ref, do,
               ebx_ref, c1_ref, c2_ref, m_ref, mslot):
    @pl.when(k == 0)
    def _():
        edge = Pn_ref[:, 0:do] / Pn_ref[:, do:do + 1]
        ebx_ref[...] = jnp.concatenate(
            [edge.astype(BF), jnp.ones((E, 8), BF)], axis=1)
        w3a = _rowdot(W3_ref[...], a2_ref[:, do:2 * do])
        s3 = jax.lax.dot_general(
            w3a, edge, (((0,), (1,)), ((), ())),
            preferred_element_type=F32)
        m3 = jnp.max(s3)
        t = s3 - m3
        c1_ref[...] = jnp.exp(t).astype(BF)
        c2_ref[...] = jnp.exp(NEG * t).astype(BF)
        m_ref[0, mslot] = m3

    q = jnp.exp((1.0 - NEG) * (s2 + m_ref[0, mslot])).astype(BF)
    B = Hb * jnp.maximum(q * c1_ref[...], c2_ref[...])
    nd = _dot(B, ebx_ref[...])
    return nd[:, :do] / nd[:, do:do + 1]


def _node_post(node, x, res_ref, g_ref, be_ref, al_ref):
    y = jnp.where(node > 0, node, jnp.exp(jnp.minimum(node, 0.0)) - 1.0)
    mu = jnp.mean(y, axis=1, keepdims=True)
    c = y - mu
    v = jnp.mean(c * c, axis=1, keepdims=True)
    xn = c * jax.lax.rsqrt(v + 1e-5) * g_ref[...] + be_ref[...]
    al = al_ref[0, 0]
    return al * xn + (1.0 - al) * _dot(x, res_ref[...])


def _hgat_kernel(x_ref, H_ref,
                 W0_ref, W20_ref, W30_ref, b0_ref, a0_ref, a20_ref,
                 ctx0_ref, res0_ref, g0_ref, be0_ref, al0_ref,
                 W1_ref, W21_ref, W31_ref, b1_ref, a1_ref, a21_ref,
                 ctx1_ref, res1_ref, g1_ref, be1_ref, al1_ref,
                 cW1_ref, cb1_ref, cW2_ref, cb2_ref,
                 out_ref,
                 x1_ref, Pn0_ref, Pn1_ref,
                 ebx0_ref, ebx1_ref, c1_ref, c2_ref,
                 m_ref):
    p = pl.program_id(0)
    k = pl.program_id(1)
    rows = pl.ds(k * RB, RB)

    @pl.when(p == 0)
    def _():
        Hb = H_ref[...].astype(BF)
        _edge_accum(k, x_ref[...], Hb, W0_ref, W20_ref, b0_ref, ctx0_ref,
                    a0_ref, a20_ref, HID, Pn0_ref, m_ref, 0)

    @pl.when(p == 1)
    def _():
        Hb = H_ref[...].astype(BF)
        x = x_ref[...]
        v2 = _rowdot(W20_ref[...], a20_ref[:, 0:HID])
        s2 = _dot(x, v2)
        node = _node_attn(k, Hb, s2, Pn0_ref, W30_ref, a20_ref,
                          HID, ebx0_ref, c1_ref, c2_ref, m_ref, 2)
        x1 = _node_post(node, x, res0_ref, g0_ref, be0_ref, al0_ref)
        x1_ref[rows, :] = x1.astype(BF)
        _edge_accum(k, x1, Hb, W1_ref, W21_ref, b1_ref, ctx1_ref,
                    a1_ref, a21_ref, OUT, Pn1_ref, m_ref, 1)

    @pl.when(p == 2)
    def _():
        Hb = H_ref[...].astype(BF)
        x1 = x1_ref[rows, :].astype(F32)
        v2 = _rowdot(W21_ref[...], a21_ref[:, 0:OUT])
        s2 = _dot(x1, v2)
        node = _node_attn(k, Hb, s2, Pn1_ref, W31_ref, a21_ref,
                          OUT, ebx1_ref, c1_ref, c2_ref, m_ref, 3)
        x2 = _node_post(node, x1, res1_ref, g1_ref, be1_ref, al1_ref)
        h = jnp.maximum(_dot(x1, cW1_ref[0:HID, :])
                        + _dot(x2, cW1_ref[HID:HID + OUT, :])
                        + cb1_ref[...], 0.0)
        out_ref[...] = _dot(h, cW2_ref[...]) + cb2_ref[...]


def _full(shape):
    nd = len(shape)
    return pl.BlockSpec(shape, lambda p, k: (0,) * nd)


def kernel(X, H, W0, W2_0, W3_0, b0, a0, a2_0, ctx0, res0, g0, be0, al0,
           W1, W2_1, W3_1, b1, a1, a2_1, ctx1, res1, g1, be1, al1,
           cW1, cb1, cW2, cb2):
    JK = HID + OUT
    out = pl.pallas_call(
        _hgat_kernel,
        grid=(3, NRB),
        in_specs=[
            pl.BlockSpec((RB, IN), lambda p, k: (jnp.where(p <= 1, k, 0), 0)),
            pl.BlockSpec((RB, E), lambda p, k: (k, 0)),
            _full((IN, HID)), _full((IN, HID)), _full((HID, HID)),
            _full((1, HID)), _full((1, 2 * HID)), _full((1, 2 * HID)),
            _full((1, HID)), _full((IN, HID)), _full((1, HID)),
            _full((1, HID)), _full((1, 1)),
            _full((HID, OUT)), _full((HID, OUT)), _full((OUT, OUT)),
            _full((1, OUT)), _full((1, 2 * OUT)), _full((1, 2 * OUT)),
            _full((1, OUT)), _full((HID, OUT)), _full((1, OUT)),
            _full((1, OUT)), _full((1, 1)),
            _full((JK, HID)), _full((1, HID)), _full((HID, OUT)),
            _full((1, OUT)),
        ],
        out_specs=pl.BlockSpec((RB, OUT),
                               lambda p, k: (jnp.where(p == 2, k, 0), 0)),
        out_shape=jax.ShapeDtypeStruct((N, OUT), F32),
        scratch_shapes=[
            pltpu.VMEM((N, HID), BF),
            pltpu.VMEM((E, HID + 8), F32),
            pltpu.VMEM((E, OUT + 8), F32),
            pltpu.VMEM((E, HID + 8), BF),
            pltpu.VMEM((E, OUT + 8), BF),
            pltpu.VMEM((1, E), BF),
            pltpu.VMEM((1, E), BF),
            pltpu.SMEM((1, 8), F32),
        ],
    )(X, H, W0, W2_0, W3_0, b0.reshape(1, HID), a0.reshape(1, 2 * HID),
      a2_0.reshape(1, 2 * HID),
      ctx0.reshape(1, HID), res0, g0.reshape(1, HID), be0.reshape(1, HID),
      al0.reshape(1, 1),
      W1, W2_1, W3_1, b1.reshape(1, OUT), a1.reshape(1, 2 * OUT),
      a2_1.reshape(1, 2 * OUT),
      ctx1.reshape(1, OUT), res1, g1.reshape(1, OUT), be1.reshape(1, OUT),
      al1.reshape(1, 1),
      cW1, cb1.reshape(1, HID), cW2, cb2.reshape(1, OUT))
    return out


# speedup vs baseline: 1.7114x; 1.0215x over previous
import jax
import jax.numpy as jnp
from jax.experimental import pallas as pl
from jax.experimental.pallas import tpu as pltpu

N, E = 10000, 2000
IN, HID, OUT = 128, 128, 64
NEG = 0.2
RB = 1000
NRB = N // RB
BF = jnp.bfloat16
F32 = jnp.float32


def _lrelu(x):
    return jnp.where(x > 0, x, NEG * x)


def _dotT(a, b):
    return jax.lax.dot_general(a, b, (((0,), (0,)), ((), ())),
                               preferred_element_type=F32)


def _dot(a, b):
    return jnp.dot(a, b, preferred_element_type=F32)


def _rowdot(a, b):
    return jax.lax.dot_general(a, b, (((1,), (1,)), ((), ())),
                               preferred_element_type=F32)


def _edge_accum(k, x, Hb, W_ref, W2_ref, b_ref, ctx_ref, a_ref, a2_ref, do,
                Pn_ref, dd_ref, m_ref, mslot):
    @pl.when(k == 0)
    def _():
        Pn_ref[...] = jnp.zeros_like(Pn_ref)
        if dd_ref is not None:
            dd_ref[...] = jnp.zeros_like(dd_ref)
        m_ref[0, mslot] = -1e30

    xt = _dot(x, W_ref[...]) + b_ref[...]
    sctx = _rowdot(ctx_ref[...], a_ref[:, 0:do])
    v1 = _rowdot(W2_ref[...], a_ref[:, do:2 * do])
    s1 = _lrelu(sctx + _dot(x, v1))

    m_old = m_ref[0, mslot]
    bmax = jnp.max(s1)
    m_new = jnp.maximum(m_old, bmax)
    w = jnp.exp(s1 - m_new)
    if dd_ref is None:
        V = jnp.concatenate(
            [(xt * w).astype(BF),
             jnp.broadcast_to(w, (w.shape[0], 8)).astype(BF)], axis=1)
    else:
        V = (xt * w).astype(BF)
        dd = jnp.sum((Hb * w.astype(BF)).astype(F32), axis=0,
                     keepdims=True)
    D = _dotT(Hb, V)

    @pl.when(bmax > m_old)
    def _():
        alpha = jnp.exp(m_old - m_new)
        Pn_ref[...] = alpha * Pn_ref[...] + D
        if dd_ref is not None:
            dd_ref[...] = alpha * dd_ref[...] + dd
        m_ref[0, mslot] = bmax

    @pl.when(bmax <= m_old)
    def _():
        Pn_ref[...] += D
        if dd_ref is not None:
            dd_ref[...] += dd


def _node_attn(k, Hb, s2, Pn_ref, dd_ref, W3_ref, a2_ref, do,
               ebx_ref, c1_ref, c2_ref, m_ref, mslot):
    @pl.when(k == 0)
    def _():
        if dd_ref is None:
            edge = Pn_ref[:, 0:do] / Pn_ref[:, do:do + 1]
            ebx_ref[...] = jnp.concatenate(
                [edge.astype(BF), jnp.ones((E, 8), BF)], axis=1)
        else:
            edge = Pn_ref[...] / jnp.transpose(dd_ref[...])
            ebx_ref[...] = edge.astype(BF)
        w3a = _rowdot(W3_ref[...], a2_ref[:, do:2 * do])
        s3 = jax.lax.dot_general(
            w3a, edge, (((0,), (1,)), ((), ())),
            preferred_element_type=F32)
        m3 = jnp.max(s3)
        t = s3 - m3
        c1_ref[...] = jnp.exp(t).astype(BF)
        c2_ref[...] = jnp.exp(NEG * t).astype(BF)
        m_ref[0, mslot] = m3

    q = jnp.exp((1.0 - NEG) * (s2 + m_ref[0, mslot])).astype(BF)
    B = Hb * jnp.maximum(q * c1_ref[...], c2_ref[...])
    nd = _dot(B, ebx_ref[...])
    if dd_ref is None:
        return nd[:, :do] / nd[:, do:do + 1]
    den = jnp.sum(B.astype(F32), axis=1, keepdims=True)
    return nd / den


def _node_post(node, x, res_ref, g_ref, be_ref, al_ref):
    y = jnp.where(node > 0, node, jnp.exp(jnp.minimum(node, 0.0)) - 1.0)
    mu = jnp.mean(y, axis=1, keepdims=True)
    c = y - mu
    v = jnp.mean(c * c, axis=1, keepdims=True)
    xn = c * jax.lax.rsqrt(v + 1e-5) * g_ref[...] + be_ref[...]
    al = al_ref[0, 0]
    return al * xn + (1.0 - al) * _dot(x, res_ref[...])


def _hgat_kernel(x_ref, H_ref,
                 W0_ref, W20_ref, W30_ref, b0_ref, a0_ref, a20_ref,
                 ctx0_ref, res0_ref, g0_ref, be0_ref, al0_ref,
                 W1_ref, W21_ref, W31_ref, b1_ref, a1_ref, a21_ref,
                 ctx1_ref, res1_ref, g1_ref, be1_ref, al1_ref,
                 cW1_ref, cb1_ref, cW2_ref, cb2_ref,
                 out_ref,
                 x1_ref, Pn0_ref, Pn1_ref, dd0_ref,
                 ebx0_ref, ebx1_ref, c1_ref, c2_ref,
                 m_ref):
    p = pl.program_id(0)
    k = pl.program_id(1)
    rows = pl.ds(k * RB, RB)

    @pl.when(p == 0)
    def _():
        Hb = H_ref[...].astype(BF)
        _edge_accum(k, x_ref[...], Hb, W0_ref, W20_ref, b0_ref, ctx0_ref,
                    a0_ref, a20_ref, HID, Pn0_ref, dd0_ref, m_ref, 0)

    @pl.when(p == 1)
    def _():
        Hb = H_ref[...].astype(BF)
        x = x_ref[...]
        v2 = _rowdot(W20_ref[...], a20_ref[:, 0:HID])
        s2 = _dot(x, v2)
        node = _node_attn(k, Hb, s2, Pn0_ref, dd0_ref, W30_ref, a20_ref,
                          HID, ebx0_ref, c1_ref, c2_ref, m_ref, 2)
        x1 = _node_post(node, x, res0_ref, g0_ref, be0_ref, al0_ref)
        x1_ref[rows, :] = x1.astype(BF)
        _edge_accum(k, x1, Hb, W1_ref, W21_ref, b1_ref, ctx1_ref,
                    a1_ref, a21_ref, OUT, Pn1_ref, None, m_ref, 1)

    @pl.when(p == 2)
    def _():
        Hb = H_ref[...].astype(BF)
        x1 = x1_ref[rows, :].astype(F32)
        v2 = _rowdot(W21_ref[...], a21_ref[:, 0:OUT])
        s2 = _dot(x1, v2)
        node = _node_attn(k, Hb, s2, Pn1_ref, None, W31_ref, a21_ref,
                          OUT, ebx1_ref, c1_ref, c2_ref, m_ref, 3)
        x2 = _node_post(node, x1, res1_ref, g1_ref, be1_ref, al1_ref)
        h = jnp.maximum(_dot(x1, cW1_ref[0:HID, :])
                        + _dot(x2, cW1_ref[HID:HID + OUT, :])
                        + cb1_ref[...], 0.0)
        out_ref[...] = _dot(h, cW2_ref[...]) + cb2_ref[...]


def _full(shape):
    nd = len(shape)
    return pl.BlockSpec(shape, lambda p, k: (0,) * nd)


def kernel(X, H, W0, W2_0, W3_0, b0, a0, a2_0, ctx0, res0, g0, be0, al0,
           W1, W2_1, W3_1, b1, a1, a2_1, ctx1, res1, g1, be1, al1,
           cW1, cb1, cW2, cb2):
    JK = HID + OUT
    out = pl.pallas_call(
        _hgat_kernel,
        grid=(3, NRB),
        in_specs=[
            pl.BlockSpec((RB, IN), lambda p, k: (jnp.where(p <= 1, k, 0), 0)),
            pl.BlockSpec((RB, E), lambda p, k: (k, 0)),
            _full((IN, HID)), _full((IN, HID)), _full((HID, HID)),
            _full((1, HID)), _full((1, 2 * HID)), _full((1, 2 * HID)),
            _full((1, HID)), _full((IN, HID)), _full((1, HID)),
            _full((1, HID)), _full((1, 1)),
            _full((HID, OUT)), _full((HID, OUT)), _full((OUT, OUT)),
            _full((1, OUT)), _full((1, 2 * OUT)), _full((1, 2 * OUT)),
            _full((1, OUT)), _full((HID, OUT)), _full((1, OUT)),
            _full((1, OUT)), _full((1, 1)),
            _full((JK, HID)), _full((1, HID)), _full((HID, OUT)),
            _full((1, OUT)),
        ],
        out_specs=pl.BlockSpec((RB, OUT),
                               lambda p, k: (jnp.where(p == 2, k, 0), 0)),
        out_shape=jax.ShapeDtypeStruct((N, OUT), F32),
        scratch_shapes=[
            pltpu.VMEM((N, HID), BF),
            pltpu.VMEM((E, HID), F32),
            pltpu.VMEM((E, OUT + 8), F32),
            pltpu.VMEM((1, E), F32),
            pltpu.VMEM((E, HID), BF),
            pltpu.VMEM((E, OUT + 8), BF),
            pltpu.VMEM((1, E), BF),
            pltpu.VMEM((1, E), BF),
            pltpu.SMEM((1, 8), F32),
        ],
    )(X, H, W0, W2_0, W3_0, b0.reshape(1, HID), a0.reshape(1, 2 * HID),
      a2_0.reshape(1, 2 * HID),
      ctx0.reshape(1, HID), res0, g0.reshape(1, HID), be0.reshape(1, HID),
      al0.reshape(1, 1),
      W1, W2_1, W3_1, b1.reshape(1, OUT), a1.reshape(1, 2 * OUT),
      a2_1.reshape(1, 2 * OUT),
      ctx1.reshape(1, OUT), res1, g1.reshape(1, OUT), be1.reshape(1, OUT),
      al1.reshape(1, 1),
      cW1, cb1.reshape(1, HID), cW2, cb2.reshape(1, OUT))
    return out
